# baseline (speedup 1.0000x reference)
import functools

import jax
import jax.numpy as jnp
from jax import lax
from jax.experimental import pallas as pl
from jax.experimental.pallas import tpu as pltpu
from jax.experimental.pallas import tpu_sc as plsc

_NBUF = 4
_LANE = 128
_BB = 512
_CHUNKS = (2048, 4096, 5120, 5120)


def _gather_rows(tpad, cats, E, b0, nb):
    B, L = cats.shape
    mesh = plsc.VectorSubcoreMesh(core_axis_name="c", subcore_axis_name="s")
    info = plsc.get_sparse_core_info()
    NC = info.num_cores
    NW = NC * info.num_subcores
    b_per_w = nb // NW

    @functools.partial(
        pl.kernel,
        out_type=jax.ShapeDtypeStruct((nb, L, E), jnp.float32),
        mesh=mesh,
        scratch_types=[
            pltpu.VMEM((b_per_w, L), jnp.int32),
            pltpu.VMEM((_NBUF, 56, _LANE), jnp.float32),
            pltpu.VMEM((_NBUF, L, E), jnp.float32),
            [pltpu.SemaphoreType.DMA] * _NBUF,
            [pltpu.SemaphoreType.DMA] * _NBUF,
        ],
        compiler_params=pltpu.CompilerParams(use_tc_tiling_on_sc=True),
    )
    def gk(tpad_hbm, cats_hbm, out_hbm, idx_v, buf, bufb, gsems, osems):
        wid = lax.axis_index("s") * NC + lax.axis_index("c")
        base = wid * b_per_w
        pltpu.sync_copy(cats_hbm.at[pl.ds(b0 + base, b_per_w)], idx_v)

        def start_gather(j, s):
            pltpu.async_copy(tpad_hbm.at[idx_v.at[j]], buf.at[s, :L], gsems[s])

        def wait_gather(j, s):
            pltpu.make_async_copy(tpad_hbm.at[idx_v.at[j]], buf.at[s, :L],
                                  gsems[s]).wait()

        def repack(s):
            def rbody(r, carry):
                for c in range(E // 16):
                    bufb[s, r, pl.ds(c * 16, 16)] = buf[s, r, pl.ds(c * 16, 16)]
                return carry

            lax.fori_loop(0, L, rbody, 0)

        def start_out(j, s):
            pltpu.async_copy(bufb.at[s], out_hbm.at[base + j], osems[s])

        def wait_out(j, s):
            pltpu.make_async_copy(bufb.at[s], out_hbm.at[base + j],
                                  osems[s]).wait()

        for s in range(_NBUF):
            start_gather(s, s)

        def body(i, carry):
            for s in range(_NBUF):
                j = i * _NBUF + s
                wait_gather(j, s)
                repack(s)
                start_out(j, s)
            for s in range(_NBUF):
                j = i * _NBUF + s
                wait_out(j, s)
                start_gather(j + _NBUF, s)
            return carry

        n_grp = b_per_w // _NBUF
        lax.fori_loop(0, n_grp - 1, body, 0)
        for s in range(_NBUF):
            j = (n_grp - 1) * _NBUF + s
            wait_gather(j, s)
            repack(s)
            start_out(j, s)
        for s in range(_NBUF):
            j = (n_grp - 1) * _NBUF + s
            wait_out(j, s)

    return gk(tpad, cats)


def _project_transpose(g, Wt, bias_col, B, b0, x_prev):
    nb, L, E = g.shape
    O = Wt.shape[1]
    blk0 = b0 // _BB

    def body(g_ref, w_ref, b_ref, *rest):
        o_ref = rest[-1]
        w = w_ref[...]
        bb = b_ref[...]
        for l in range(L):
            x = g_ref[:, l, :]
            y = lax.dot_general(w, x, (((0,), (1,)), ((), ())),
                                preferred_element_type=jnp.float32)
            o_ref[l] = y + bb

    in_specs = [
        pl.BlockSpec((_BB, L, E), lambda i: (i, 0, 0)),
        pl.BlockSpec((E, O), lambda i: (0, 0)),
        pl.BlockSpec((O, 1), lambda i: (0, 0)),
    ]
    args = [g, Wt, bias_col]
    aliases = {}
    if x_prev is not None:
        in_specs.append(pl.BlockSpec(memory_space=pl.ANY))
        args.append(x_prev)
        aliases = {3: 0}

    return pl.pallas_call(
        body,
        grid=(nb // _BB,),
        in_specs=in_specs,
        out_specs=pl.BlockSpec((L, O, _BB), lambda i: (0, 0, blk0 + i)),
        out_shape=jax.ShapeDtypeStruct((L, O, B), jnp.float32),
        input_output_aliases=aliases,
    )(*args)


def kernel(categories, table, W, b):
    O, E = W.shape
    B, L = categories.shape
    tpad = jnp.pad(table, ((0, 0), (0, _LANE - E)))
    cats = categories.astype(jnp.int32)
    Wt = W.T
    bias_col = b.reshape(O, 1)

    x = None
    b0s = [sum(_CHUNKS[:c]) for c in range(len(_CHUNKS))]
    gs = [_gather_rows(tpad, cats, E, b0s[c], _CHUNKS[c])
          for c in range(len(_CHUNKS))]
    for c in range(len(_CHUNKS)):
        x = _project_transpose(gs[c], Wt, bias_col, B, b0s[c], x)
    return jnp.transpose(x, (2, 0, 1))


# speedup vs baseline: 1.0053x; 1.0053x over previous
import functools

import jax
import jax.numpy as jnp
from jax import lax
from jax.experimental import pallas as pl
from jax.experimental.pallas import tpu as pltpu
from jax.experimental.pallas import tpu_sc as plsc

_NBUF = 4
_LANE = 128
_BB = 512
_CHUNKS = (2048,) * 8


def _gather_rows(tpad, cats, E, b0, nb):
    B, L = cats.shape
    mesh = plsc.VectorSubcoreMesh(core_axis_name="c", subcore_axis_name="s")
    info = plsc.get_sparse_core_info()
    NC = info.num_cores
    NW = NC * info.num_subcores
    b_per_w = nb // NW

    @functools.partial(
        pl.kernel,
        out_type=jax.ShapeDtypeStruct((nb, L, E), jnp.float32),
        mesh=mesh,
        scratch_types=[
            pltpu.VMEM((b_per_w, L), jnp.int32),
            pltpu.VMEM((_NBUF, 56, _LANE), jnp.float32),
            pltpu.VMEM((_NBUF, L, E), jnp.float32),
            [pltpu.SemaphoreType.DMA] * _NBUF,
            [pltpu.SemaphoreType.DMA] * _NBUF,
        ],
        compiler_params=pltpu.CompilerParams(use_tc_tiling_on_sc=True),
    )
    def gk(tpad_hbm, cats_hbm, out_hbm, idx_v, buf, bufb, gsems, osems):
        wid = lax.axis_index("s") * NC + lax.axis_index("c")
        base = wid * b_per_w
        pltpu.sync_copy(cats_hbm.at[pl.ds(b0 + base, b_per_w)], idx_v)

        def start_gather(j, s):
            pltpu.async_copy(tpad_hbm.at[idx_v.at[j]], buf.at[s, :L], gsems[s])

        def wait_gather(j, s):
            pltpu.make_async_copy(tpad_hbm.at[idx_v.at[j]], buf.at[s, :L],
                                  gsems[s]).wait()

        def repack(s):
            def rbody(r, carry):
                for c in range(E // 16):
                    bufb[s, r, pl.ds(c * 16, 16)] = buf[s, r, pl.ds(c * 16, 16)]
                return carry

            lax.fori_loop(0, L, rbody, 0)

        def start_out(j, s):
            pltpu.async_copy(bufb.at[s], out_hbm.at[base + j], osems[s])

        def wait_out(j, s):
            pltpu.make_async_copy(bufb.at[s], out_hbm.at[base + j],
                                  osems[s]).wait()

        for s in range(_NBUF):
            start_gather(s, s)

        def body(i, carry):
            for s in range(_NBUF):
                j = i * _NBUF + s
                wait_gather(j, s)
                repack(s)
                start_out(j, s)
            for s in range(_NBUF):
                j = i * _NBUF + s
                wait_out(j, s)
                start_gather(j + _NBUF, s)
            return carry

        n_grp = b_per_w // _NBUF
        lax.fori_loop(0, n_grp - 1, body, 0)
        for s in range(_NBUF):
            j = (n_grp - 1) * _NBUF + s
            wait_gather(j, s)
            repack(s)
            start_out(j, s)
        for s in range(_NBUF):
            j = (n_grp - 1) * _NBUF + s
            wait_out(j, s)

    return gk(tpad, cats)


def _project_transpose(g, Wt, bias_col, B, b0, x_prev):
    nb, L, E = g.shape
    O = Wt.shape[1]
    blk0 = b0 // _BB

    def body(g_ref, w_ref, b_ref, *rest):
        o_ref = rest[-1]
        w = w_ref[...]
        bb = b_ref[...]
        for l in range(L):
            x = g_ref[:, l, :]
            y = lax.dot_general(w, x, (((0,), (1,)), ((), ())),
                                preferred_element_type=jnp.float32)
            o_ref[l] = y + bb

    in_specs = [
        pl.BlockSpec((_BB, L, E), lambda i: (i, 0, 0)),
        pl.BlockSpec((E, O), lambda i: (0, 0)),
        pl.BlockSpec((O, 1), lambda i: (0, 0)),
    ]
    args = [g, Wt, bias_col]
    aliases = {}
    if x_prev is not None:
        in_specs.append(pl.BlockSpec(memory_space=pl.ANY))
        args.append(x_prev)
        aliases = {3: 0}

    return pl.pallas_call(
        body,
        grid=(nb // _BB,),
        in_specs=in_specs,
        out_specs=pl.BlockSpec((L, O, _BB), lambda i: (0, 0, blk0 + i)),
        out_shape=jax.ShapeDtypeStruct((L, O, B), jnp.float32),
        input_output_aliases=aliases,
    )(*args)


def kernel(categories, table, W, b):
    O, E = W.shape
    B, L = categories.shape
    tpad = jnp.pad(table, ((0, 0), (0, _LANE - E)))
    cats = categories.astype(jnp.int32)
    Wt = W.T
    bias_col = b.reshape(O, 1)

    x = None
    b0s = [sum(_CHUNKS[:c]) for c in range(len(_CHUNKS))]
    gs = [_gather_rows(tpad, cats, E, b0s[c], _CHUNKS[c])
          for c in range(len(_CHUNKS))]
    for c in range(len(_CHUNKS)):
        x = _project_transpose(gs[c], Wt, bias_col, B, b0s[c], x)
    return jnp.transpose(x, (2, 0, 1))


# speedup vs baseline: 1.0119x; 1.0065x over previous
import functools

import jax
import jax.numpy as jnp
from jax import lax
from jax.experimental import pallas as pl
from jax.experimental.pallas import tpu as pltpu
from jax.experimental.pallas import tpu_sc as plsc

_NBUF = 8
_LANE = 128
_BB = 512
_CHUNKS = (4096,) * 4


def _gather_rows(tpad, cats, E, b0, nb):
    B, L = cats.shape
    mesh = plsc.VectorSubcoreMesh(core_axis_name="c", subcore_axis_name="s")
    info = plsc.get_sparse_core_info()
    NC = info.num_cores
    NW = NC * info.num_subcores
    b_per_w = nb // NW

    @functools.partial(
        pl.kernel,
        out_type=jax.ShapeDtypeStruct((nb, L, E), jnp.float32),
        mesh=mesh,
        scratch_types=[
            pltpu.VMEM((b_per_w, L), jnp.int32),
            pltpu.VMEM((_NBUF, 56, _LANE), jnp.float32),
            pltpu.VMEM((_NBUF, L, E), jnp.float32),
            [pltpu.SemaphoreType.DMA] * _NBUF,
            [pltpu.SemaphoreType.DMA] * _NBUF,
        ],
        compiler_params=pltpu.CompilerParams(use_tc_tiling_on_sc=True),
    )
    def gk(tpad_hbm, cats_hbm, out_hbm, idx_v, buf, bufb, gsems, osems):
        wid = lax.axis_index("s") * NC + lax.axis_index("c")
        base = wid * b_per_w
        pltpu.sync_copy(cats_hbm.at[pl.ds(b0 + base, b_per_w)], idx_v)

        def start_gather(j, s):
            pltpu.async_copy(tpad_hbm.at[idx_v.at[j]], buf.at[s, :L], gsems[s])

        def wait_gather(j, s):
            pltpu.make_async_copy(tpad_hbm.at[idx_v.at[j]], buf.at[s, :L],
                                  gsems[s]).wait()

        def repack(s):
            def rbody(r, carry):
                for c in range(E // 16):
                    bufb[s, r, pl.ds(c * 16, 16)] = buf[s, r, pl.ds(c * 16, 16)]
                return carry

            lax.fori_loop(0, L, rbody, 0)

        def start_out(j, s):
            pltpu.async_copy(bufb.at[s], out_hbm.at[base + j], osems[s])

        def wait_out(j, s):
            pltpu.make_async_copy(bufb.at[s], out_hbm.at[base + j],
                                  osems[s]).wait()

        for s in range(_NBUF):
            start_gather(s, s)

        def body(i, carry):
            for s in range(_NBUF):
                j = i * _NBUF + s
                wait_gather(j, s)
                repack(s)
                start_out(j, s)
            for s in range(_NBUF):
                j = i * _NBUF + s
                wait_out(j, s)
                start_gather(j + _NBUF, s)
            return carry

        n_grp = b_per_w // _NBUF
        lax.fori_loop(0, n_grp - 1, body, 0)
        for s in range(_NBUF):
            j = (n_grp - 1) * _NBUF + s
            wait_gather(j, s)
            repack(s)
            start_out(j, s)
        for s in range(_NBUF):
            j = (n_grp - 1) * _NBUF + s
            wait_out(j, s)

    return gk(tpad, cats)


def _project_transpose(g, Wt, bias_col, B, b0, x_prev):
    nb, L, E = g.shape
    O = Wt.shape[1]
    blk0 = b0 // _BB

    def body(g_ref, w_ref, b_ref, *rest):
        o_ref = rest[-1]
        w = w_ref[...]
        bb = b_ref[...]
        for l in range(L):
            x = g_ref[:, l, :]
            y = lax.dot_general(w, x, (((0,), (1,)), ((), ())),
                                preferred_element_type=jnp.float32)
            o_ref[l] = y + bb

    in_specs = [
        pl.BlockSpec((_BB, L, E), lambda i: (i, 0, 0)),
        pl.BlockSpec((E, O), lambda i: (0, 0)),
        pl.BlockSpec((O, 1), lambda i: (0, 0)),
    ]
    args = [g, Wt, bias_col]
    aliases = {}
    if x_prev is not None:
        in_specs.append(pl.BlockSpec(memory_space=pl.ANY))
        args.append(x_prev)
        aliases = {3: 0}

    return pl.pallas_call(
        body,
        grid=(nb // _BB,),
        in_specs=in_specs,
        out_specs=pl.BlockSpec((L, O, _BB), lambda i: (0, 0, blk0 + i)),
        out_shape=jax.ShapeDtypeStruct((L, O, B), jnp.float32),
        input_output_aliases=aliases,
    )(*args)


def kernel(categories, table, W, b):
    O, E = W.shape
    B, L = categories.shape
    tpad = jnp.pad(table, ((0, 0), (0, _LANE - E)))
    cats = categories.astype(jnp.int32)
    Wt = W.T
    bias_col = b.reshape(O, 1)

    x = None
    b0s = [sum(_CHUNKS[:c]) for c in range(len(_CHUNKS))]
    gs = [_gather_rows(tpad, cats, E, b0s[c], _CHUNKS[c])
          for c in range(len(_CHUNKS))]
    for c in range(len(_CHUNKS)):
        x = _project_transpose(gs[c], Wt, bias_col, B, b0s[c], x)
    return jnp.transpose(x, (2, 0, 1))
